# baseline (speedup 1.0000x reference)
import functools

import jax
import jax.numpy as jnp
from jax import lax
from jax.experimental import pallas as pl
from jax.experimental.pallas import tpu as pltpu
from jax.experimental.pallas import tpu_sc as plsc

_LANES = 16


def _sc_geometry():
    try:
        info = plsc.get_sparse_core_info()
        return info.num_cores, info.num_subcores
    except Exception:
        return 2, 16


def kernel(inputs, token_table, pos_table):
    batch, seq = inputs.shape
    vocab, emb = token_table.shape
    n = batch * seq

    nc, ns = _sc_geometry()
    nw = nc * ns
    rows_per_w = batch // nw

    idx_flat = inputs.reshape(n).astype(jnp.int32)

    mesh = plsc.VectorSubcoreMesh(core_axis_name="c", subcore_axis_name="s")

    @functools.partial(
        pl.kernel,
        out_type=jax.ShapeDtypeStruct((batch, seq, emb), jnp.float32),
        mesh=mesh,
        scratch_types=[
            pltpu.VMEM((seq,), jnp.int32),
            pltpu.VMEM((seq, emb), jnp.float32),
            pltpu.VMEM((seq, emb), jnp.float32),
            pltpu.SemaphoreType.DMA,
        ],
        compiler_params=pltpu.CompilerParams(use_tc_tiling_on_sc=False),
    )
    def sc_kernel(idx_hbm, tok_hbm, pos_hbm, out_hbm, idx_v, rows_v, pos_v, sem):
        wid = lax.axis_index("s") * nc + lax.axis_index("c")
        row0 = wid * rows_per_w
        pltpu.sync_copy(pos_hbm, pos_v)

        def row_body(q, carry):
            b = row0 + q
            pltpu.sync_copy(idx_hbm.at[pl.ds(b * seq, seq)], idx_v)
            pltpu.async_copy(tok_hbm.at[idx_v], rows_v, sem).wait()

            def s_body(s, carry2):
                for g in range(emb // _LANES):
                    sl = pl.ds(g * _LANES, _LANES)
                    plsc.addupdate(rows_v.at[s, sl], pos_v[s, sl])
                return carry2

            lax.fori_loop(0, seq, s_body, 0)
            pltpu.sync_copy(rows_v, out_hbm.at[b])
            return carry

        lax.fori_loop(0, rows_per_w, row_body, 0)

    return sc_kernel(idx_flat, token_table, pos_table)


# speedup vs baseline: 1.0785x; 1.0785x over previous
import functools

import jax
import jax.numpy as jnp
from jax import lax
from jax.experimental import pallas as pl
from jax.experimental.pallas import tpu as pltpu
from jax.experimental.pallas import tpu_sc as plsc

_LANES = 16


def _sc_geometry():
    try:
        info = plsc.get_sparse_core_info()
        return info.num_cores, info.num_subcores
    except Exception:
        return 2, 16


def kernel(inputs, token_table, pos_table):
    batch, seq = inputs.shape
    vocab, emb = token_table.shape
    n = batch * seq

    nc, ns = _sc_geometry()
    nw = nc * ns
    rows_per_w = batch // nw

    idx_flat = inputs.reshape(n).astype(jnp.int32)
    tok128 = jnp.pad(token_table, ((0, 0), (0, 128 - emb)))

    mesh = plsc.VectorSubcoreMesh(core_axis_name="c", subcore_axis_name="s")

    @functools.partial(
        pl.kernel,
        out_type=jax.ShapeDtypeStruct((batch, seq, emb), jnp.float32),
        mesh=mesh,
        scratch_types=[
            pltpu.VMEM((seq,), jnp.int32),
            pltpu.VMEM((seq, 128), jnp.float32),
            pltpu.VMEM((seq, emb), jnp.float32),
            pltpu.VMEM((seq, emb), jnp.float32),
            pltpu.SemaphoreType.DMA,
        ],
        compiler_params=pltpu.CompilerParams(use_tc_tiling_on_sc=True),
    )
    def sc_kernel(idx_hbm, tok_hbm, pos_hbm, out_hbm, idx_v, rows_v, pos_v,
                  out_v, sem):
        wid = lax.axis_index("s") * nc + lax.axis_index("c")
        row0 = wid * rows_per_w
        pltpu.sync_copy(pos_hbm, pos_v)

        def row_body(q, carry):
            b = row0 + q
            pltpu.sync_copy(idx_hbm.at[pl.ds(b * seq, seq)], idx_v)
            pltpu.async_copy(tok_hbm.at[idx_v], rows_v, sem).wait()

            def s_body(s, carry2):
                for g in range(emb // _LANES):
                    sl = pl.ds(g * _LANES, _LANES)
                    out_v[s, sl] = rows_v[s, sl] + pos_v[s, sl]
                return carry2

            lax.fori_loop(0, seq, s_body, 0)
            pltpu.sync_copy(out_v, out_hbm.at[b])
            return carry

        lax.fori_loop(0, rows_per_w, row_body, 0)

    return sc_kernel(idx_flat, tok128, pos_table)
